# baseline (speedup 1.0000x reference)
import functools

import jax
import jax.numpy as jnp
from jax import lax
from jax.experimental import pallas as pl
from jax.experimental.pallas import tpu as pltpu
from jax.experimental.pallas import tpu_sc as plsc

N = 10000
E = 320000
D = 128
R_BASIS = 8
H = 64
INV_AVG = 1.0 / 32.0

NC = 2
NS = 16
NW = NC * NS
L = 16

CHUNK = 88

EH = E // 2
SLICES = ((EH, 3200, 56, 72), (EH, 3200, 56, 72))
RSEG = 32

ROWS_PER_W = 624
ZROWS = 48
NZ = ROWS_PER_W // ZROWS
REM_ROW0 = NS * ROWS_PER_W
REM_ROWS = N - REM_ROW0

_DN0 = (((0,), (0,)), ((), ()))


def _coeff_body(ev_ref, rad_ref, w1_ref, w2_ref, wsh_ref, out_ref):
    ev = ev_ref[...]
    r = jnp.sqrt(jnp.sum(ev * ev, axis=0, keepdims=True))
    unit = ev / (r + 1e-9)
    ang = (wsh_ref[0, 0]
           + unit[0:1, :] * wsh_ref[0, 1]
           + unit[1:2, :] * wsh_ref[0, 2]
           + unit[2:3, :] * wsh_ref[0, 3])
    hT = lax.dot_general(w1_ref[...], rad_ref[...], _DN0,
                         preferred_element_type=jnp.float32)
    hT = hT * jax.nn.sigmoid(hT) * (ang * INV_AVG)
    out_ref[...] = lax.dot_general(hT, w2_ref[...], _DN0,
                                   preferred_element_type=jnp.float32)


def _coeff(ev_t, rad_t, W1, W2, wsh, BE):
    ne = ev_t.shape[1]
    grid = (ne // BE,)
    return pl.pallas_call(
        _coeff_body,
        grid=grid,
        in_specs=[
            pl.BlockSpec((3, BE), lambda i: (0, i)),
            pl.BlockSpec((R_BASIS, BE), lambda i: (0, i)),
            pl.BlockSpec((R_BASIS, H), lambda i: (0, 0)),
            pl.BlockSpec((H, D), lambda i: (0, 0)),
            pl.BlockSpec((1, 4), lambda i: (0, 0)),
        ],
        out_specs=pl.BlockSpec((BE, D), lambda i: (i, 0)),
        out_shape=jax.ShapeDtypeStruct((ne, D), jnp.float32),
    )(ev_t, rad_t, W1, W2, wsh)


def _mul_rows(dst, src_c, nrows):
    @plsc.parallel_loop(0, nrows, unroll=4)
    def _mul(r):
        for j in range(D // L):
            sl = pl.ds(j * L, L)
            dst[r, sl] = dst[r, sl] * src_c[r, sl]


def _make_sc_body(EPW, NCH, TAIL):
  def _sc_body(node_ref, send_ref, recv_ref, recv3_ref, coeff_ref, out_ref,
               acc, sidx, rall, crow, nrow, ridx_t,
               semi, semc, semg, sems):
    cid = lax.axis_index("c")
    sid = lax.axis_index("s")
    wid = cid * NS + sid
    ebase = wid * EPW

    def _issue_sidx(c, s2):
        pltpu.async_copy(send_ref.at[pl.ds(ebase + c * CHUNK, CHUNK)],
                         sidx[s2], semi[s2])

    def _issue_main(c, s2):
        pltpu.make_async_copy(send_ref.at[pl.ds(0, CHUNK)], sidx[s2],
                              semi[s2]).wait()
        pltpu.async_copy(node_ref.at[sidx[s2]], nrow[s2], semg[s2])
        pltpu.async_copy(coeff_ref.at[pl.ds(ebase + c * CHUNK, CHUNK)],
                         crow[s2], semc[s2])

    _issue_sidx(0, 0)
    _issue_sidx(1, 1)
    pltpu.async_copy(recv3_ref.at[wid, pl.ds(0, RSEG)], rall, semg[1])

    @pl.loop(0, ZROWS)
    def _zero_rows(i):
        for j in range(D // L):
            nrow[1][i, pl.ds(j * L, L)] = jnp.zeros((L,), jnp.float32)

    for k in range(NZ):
        pltpu.async_copy(nrow[1].at[pl.ds(0, ZROWS)],
                         acc.at[pl.ds(sid * ROWS_PER_W + k * ZROWS, ZROWS)],
                         semc[0])

    @pl.when(sid == NS - 1)
    def _zero_rem():
        pltpu.async_copy(nrow[1].at[pl.ds(0, REM_ROWS)],
                         acc.at[pl.ds(REM_ROW0, REM_ROWS)], semc[0])

    for k in range(NZ):
        pltpu.make_async_copy(
            nrow[1].at[pl.ds(0, ZROWS)],
            acc.at[pl.ds(sid * ROWS_PER_W + k * ZROWS, ZROWS)],
            semc[0]).wait()

    @pl.when(sid == NS - 1)
    def _zero_rem_wait():
        pltpu.make_async_copy(nrow[1].at[pl.ds(0, REM_ROWS)],
                              acc.at[pl.ds(REM_ROW0, REM_ROWS)],
                              semc[0]).wait()

    pltpu.make_async_copy(recv3_ref.at[wid, pl.ds(0, RSEG)], rall,
                          semg[1]).wait()

    plsc.subcore_barrier()

    _issue_main(0, 0)

    def _chunk_body(c, b4, rrow):
        b = b4 % 2
        o = 1 - b

        @pl.when(c >= 1)
        def _drain():
            pltpu.make_async_copy(nrow[o], acc.at[rall.at[0]],
                                  sems[o]).wait()

        @pl.when(c + 1 < NCH)
        def _pf_main():
            _issue_main(c + 1, o)

        pltpu.make_async_copy(node_ref.at[sidx[b]], nrow[b],
                              semg[b]).wait()
        pltpu.make_async_copy(coeff_ref.at[pl.ds(0, CHUNK)], crow[b],
                              semc[b]).wait()

        _mul_rows(nrow[b], crow[b], CHUNK)

        pltpu.async_copy(nrow[b], acc.at[rall.at[rrow]], sems[b], add=True)

        @pl.when(c + 2 < NCH)
        def _pf_sidx():
            _issue_sidx(c + 2, b)

    NCH_A = min(RSEG, NCH)

    @pl.loop(0, NCH_A, step=4)
    def _chunks_a(i):
        for b4 in range(4):
            _chunk_body(i + b4, b4, i + b4)

    if NCH > NCH_A:
        pltpu.sync_copy(recv3_ref.at[wid, pl.ds(NCH_A, NCH - NCH_A)],
                        rall.at[pl.ds(0, NCH - NCH_A)])

        @pl.loop(0, NCH - NCH_A, step=4)
        def _chunks_b(i):
            for b4 in range(4):
                _chunk_body(NCH_A + i + b4, b4, i + b4)

    pltpu.make_async_copy(nrow[(NCH - 1) % 2], acc.at[rall.at[0]],
                          sems[(NCH - 1) % 2]).wait()

    if TAIL:
        te0 = ebase + NCH * CHUNK
        pltpu.sync_copy(send_ref.at[pl.ds(te0, TAIL)],
                        sidx[0].at[pl.ds(0, TAIL)])
        pltpu.sync_copy(recv_ref.at[pl.ds(te0, TAIL)], ridx_t)
        pltpu.sync_copy(coeff_ref.at[pl.ds(te0, TAIL)],
                        crow[0].at[pl.ds(0, TAIL)])
        pltpu.async_copy(node_ref.at[sidx[0].at[pl.ds(0, TAIL)]],
                         nrow[0].at[pl.ds(0, TAIL)], semg[0])
        pltpu.make_async_copy(node_ref.at[sidx[0].at[pl.ds(0, TAIL)]],
                              nrow[0].at[pl.ds(0, TAIL)], semg[0]).wait()
        _mul_rows(nrow[0], crow[0], TAIL)
        pltpu.sync_copy(nrow[0].at[pl.ds(0, TAIL)], acc.at[ridx_t], add=True)

    plsc.subcore_barrier()

    row0 = sid * ROWS_PER_W
    pltpu.sync_copy(acc.at[pl.ds(row0, ROWS_PER_W)],
                    out_ref.at[cid, pl.ds(row0, ROWS_PER_W)])

    @pl.when(sid == NS - 1)
    def _writeback_rem():
        pltpu.sync_copy(acc.at[pl.ds(REM_ROW0, REM_ROWS)],
                        out_ref.at[cid, pl.ds(REM_ROW0, REM_ROWS)])

  return _sc_body


def _sc_scatter(node_feats, senders, receivers, recv3, coeff, nch, tail):
    epw = senders.shape[0] // NW
    mesh = plsc.VectorSubcoreMesh(core_axis_name="c", subcore_axis_name="s")
    kern = pl.kernel(
        _make_sc_body(epw, nch, tail),
        out_type=jax.ShapeDtypeStruct((NC, N, D), jnp.float32),
        mesh=mesh,
        scratch_types=[
            pltpu.VMEM_SHARED((N, D), jnp.float32),
            [pltpu.VMEM((CHUNK,), jnp.int32) for _ in range(2)],
            pltpu.VMEM((min(RSEG, nch), CHUNK), jnp.int32),
            [pltpu.VMEM((CHUNK, D), jnp.float32) for _ in range(2)],
            [pltpu.VMEM((CHUNK, D), jnp.float32) for _ in range(2)],
            pltpu.VMEM((max(tail, 8),), jnp.int32),
            [pltpu.SemaphoreType.DMA for _ in range(2)],
            [pltpu.SemaphoreType.DMA for _ in range(2)],
            [pltpu.SemaphoreType.DMA for _ in range(2)],
            [pltpu.SemaphoreType.DMA for _ in range(2)],
        ],
    )
    return kern(node_feats, senders, receivers, recv3, coeff)


def _combine_body(p0_ref, p1_ref, out_ref):
    out_ref[...] = (p0_ref[0] + p0_ref[1]) + (p1_ref[0] + p1_ref[1])


def _combine(part0, part1):
    BN = 2000
    return pl.pallas_call(
        _combine_body,
        grid=(N // BN,),
        in_specs=[pl.BlockSpec((NC, BN, D), lambda i: (0, i, 0)),
                  pl.BlockSpec((NC, BN, D), lambda i: (0, i, 0))],
        out_specs=pl.BlockSpec((BN, D), lambda i: (i, 0)),
        out_shape=jax.ShapeDtypeStruct((N, D), jnp.float32),
    )(part0, part1)


def kernel(edge_vectors, node_feats, radial_embeddings, senders, receivers,
           W1, W2, w_sh):
    ev_t = edge_vectors.T
    rad_t = radial_embeddings.T
    wsh = w_sh.reshape(1, 4)
    senders = senders.astype(jnp.int32)
    receivers = receivers.astype(jnp.int32)

    parts = []
    lo = 0
    for ne, be, nch, tail in SLICES:
        hi = lo + ne
        epw = ne // NW
        recs = receivers[lo:hi].reshape(NW, epw)
        recv3 = recs[:, :nch * CHUNK].reshape(NW, nch, CHUNK)
        coeff = _coeff(ev_t[:, lo:hi], rad_t[:, lo:hi], W1, W2, wsh, be)
        parts.append(_sc_scatter(node_feats, senders[lo:hi],
                                 receivers[lo:hi], recv3, coeff, nch, tail))
        lo = hi
    return _combine(parts[0], parts[1])


# speedup vs baseline: 1.0302x; 1.0302x over previous
import functools

import jax
import jax.numpy as jnp
from jax import lax
from jax.experimental import pallas as pl
from jax.experimental.pallas import tpu as pltpu
from jax.experimental.pallas import tpu_sc as plsc

N = 10000
E = 320000
D = 128
R_BASIS = 8
H = 64
INV_AVG = 1.0 / 32.0

NC = 2
NS = 16
NW = NC * NS
L = 16

CHUNK = 96

EH = E // 2
SLICES = ((EH, 3200, 52, 8), (EH, 3200, 52, 8))

ROWS_PER_W = 624
ZROWS = 48
NZ = ROWS_PER_W // ZROWS
REM_ROW0 = NS * ROWS_PER_W
REM_ROWS = N - REM_ROW0

_DN0 = (((0,), (0,)), ((), ()))


def _coeff_body(ev_ref, rad_ref, w1_ref, w2_ref, wsh_ref, out_ref):
    ev = ev_ref[...]
    r = jnp.sqrt(jnp.sum(ev * ev, axis=0, keepdims=True))
    unit = ev / (r + 1e-9)
    ang = (wsh_ref[0, 0]
           + unit[0:1, :] * wsh_ref[0, 1]
           + unit[1:2, :] * wsh_ref[0, 2]
           + unit[2:3, :] * wsh_ref[0, 3])
    hT = lax.dot_general(w1_ref[...], rad_ref[...], _DN0,
                         preferred_element_type=jnp.float32)
    hT = hT * jax.nn.sigmoid(hT) * (ang * INV_AVG)
    out_ref[...] = lax.dot_general(hT, w2_ref[...], _DN0,
                                   preferred_element_type=jnp.float32)


def _coeff(ev_t, rad_t, W1, W2, wsh, BE):
    ne = ev_t.shape[1]
    grid = (ne // BE,)
    return pl.pallas_call(
        _coeff_body,
        grid=grid,
        in_specs=[
            pl.BlockSpec((3, BE), lambda i: (0, i)),
            pl.BlockSpec((R_BASIS, BE), lambda i: (0, i)),
            pl.BlockSpec((R_BASIS, H), lambda i: (0, 0)),
            pl.BlockSpec((H, D), lambda i: (0, 0)),
            pl.BlockSpec((1, 4), lambda i: (0, 0)),
        ],
        out_specs=pl.BlockSpec((BE, D), lambda i: (i, 0)),
        out_shape=jax.ShapeDtypeStruct((ne, D), jnp.float32),
    )(ev_t, rad_t, W1, W2, wsh)


def _mul_rows(dst, src_c, nrows):
    @plsc.parallel_loop(0, nrows, unroll=4)
    def _mul(r):
        for j in range(D // L):
            sl = pl.ds(j * L, L)
            dst[r, sl] = dst[r, sl] * src_c[r, sl]


def _make_sc_body(EPW, NCH, TAIL):
  def _sc_body(node_ref, send_ref, recv_ref, coeff_ref, out_ref,
               acc, sidx, ridx, crow, nrow, ridx_t,
               semi, semr, semc, semg, sems):
    cid = lax.axis_index("c")
    sid = lax.axis_index("s")
    wid = cid * NS + sid
    ebase = wid * EPW

    def _issue_sidx(c, s2):
        pltpu.async_copy(send_ref.at[pl.ds(ebase + c * CHUNK, CHUNK)],
                         sidx[s2], semi[s2])

    def _issue_ridx(c, s4):
        pltpu.async_copy(recv_ref.at[pl.ds(ebase + c * CHUNK, CHUNK)],
                         ridx[s4], semr[s4])

    def _issue_main(c, s2):
        pltpu.make_async_copy(send_ref.at[pl.ds(0, CHUNK)], sidx[s2],
                              semi[s2]).wait()
        pltpu.async_copy(node_ref.at[sidx[s2]], nrow[s2], semg[s2])
        pltpu.async_copy(coeff_ref.at[pl.ds(ebase + c * CHUNK, CHUNK)],
                         crow[s2], semc[s2])

    _issue_sidx(0, 0)
    _issue_sidx(1, 1)
    _issue_ridx(0, 0)
    _issue_ridx(1, 1)
    _issue_ridx(2, 2)

    @pl.loop(0, ZROWS)
    def _zero_rows(i):
        for j in range(D // L):
            nrow[1][i, pl.ds(j * L, L)] = jnp.zeros((L,), jnp.float32)

    for k in range(NZ):
        pltpu.async_copy(nrow[1].at[pl.ds(0, ZROWS)],
                         acc.at[pl.ds(sid * ROWS_PER_W + k * ZROWS, ZROWS)],
                         semc[0])

    @pl.when(sid == NS - 1)
    def _zero_rem():
        pltpu.async_copy(nrow[1].at[pl.ds(0, REM_ROWS)],
                         acc.at[pl.ds(REM_ROW0, REM_ROWS)], semc[0])

    for k in range(NZ):
        pltpu.make_async_copy(
            nrow[1].at[pl.ds(0, ZROWS)],
            acc.at[pl.ds(sid * ROWS_PER_W + k * ZROWS, ZROWS)],
            semc[0]).wait()

    @pl.when(sid == NS - 1)
    def _zero_rem_wait():
        pltpu.make_async_copy(nrow[1].at[pl.ds(0, REM_ROWS)],
                              acc.at[pl.ds(REM_ROW0, REM_ROWS)],
                              semc[0]).wait()

    plsc.subcore_barrier()

    _issue_main(0, 0)

    @pl.loop(0, NCH, step=4)
    def _chunks(i):
        for b4 in range(4):
            c = i + b4
            b = b4 % 2
            o = 1 - b

            @pl.when(c >= 1)
            def _drain():
                pltpu.make_async_copy(nrow[o], acc.at[ridx[b4]],
                                      sems[o]).wait()

            @pl.when(c + 1 < NCH)
            def _pf_main():
                _issue_main(c + 1, o)

            pltpu.make_async_copy(node_ref.at[sidx[b]], nrow[b],
                                  semg[b]).wait()
            pltpu.make_async_copy(coeff_ref.at[pl.ds(0, CHUNK)], crow[b],
                                  semc[b]).wait()

            _mul_rows(nrow[b], crow[b], CHUNK)

            pltpu.make_async_copy(recv_ref.at[pl.ds(0, CHUNK)], ridx[b4],
                                  semr[b4]).wait()
            pltpu.async_copy(nrow[b], acc.at[ridx[b4]], sems[b], add=True)

            @pl.when(c + 2 < NCH)
            def _pf_sidx():
                _issue_sidx(c + 2, b)

            @pl.when(c + 3 < NCH)
            def _pf_ridx():
                _issue_ridx(c + 3, (b4 + 3) % 4)

    pltpu.make_async_copy(nrow[(NCH - 1) % 2], acc.at[ridx[(NCH - 1) % 4]],
                          sems[(NCH - 1) % 2]).wait()

    if TAIL:
        te0 = ebase + NCH * CHUNK
        pltpu.sync_copy(send_ref.at[pl.ds(te0, TAIL)],
                        sidx[0].at[pl.ds(0, TAIL)])
        pltpu.sync_copy(recv_ref.at[pl.ds(te0, TAIL)], ridx_t)
        pltpu.sync_copy(coeff_ref.at[pl.ds(te0, TAIL)],
                        crow[0].at[pl.ds(0, TAIL)])
        pltpu.async_copy(node_ref.at[sidx[0].at[pl.ds(0, TAIL)]],
                         nrow[0].at[pl.ds(0, TAIL)], semg[0])
        pltpu.make_async_copy(node_ref.at[sidx[0].at[pl.ds(0, TAIL)]],
                              nrow[0].at[pl.ds(0, TAIL)], semg[0]).wait()
        _mul_rows(nrow[0], crow[0], TAIL)
        pltpu.sync_copy(nrow[0].at[pl.ds(0, TAIL)], acc.at[ridx_t], add=True)

    plsc.subcore_barrier()

    row0 = sid * ROWS_PER_W
    pltpu.sync_copy(acc.at[pl.ds(row0, ROWS_PER_W)],
                    out_ref.at[cid, pl.ds(row0, ROWS_PER_W)])

    @pl.when(sid == NS - 1)
    def _writeback_rem():
        pltpu.sync_copy(acc.at[pl.ds(REM_ROW0, REM_ROWS)],
                        out_ref.at[cid, pl.ds(REM_ROW0, REM_ROWS)])

  return _sc_body


def _sc_scatter(node_feats, senders, receivers, coeff, nch, tail):
    epw = senders.shape[0] // NW
    mesh = plsc.VectorSubcoreMesh(core_axis_name="c", subcore_axis_name="s")
    kern = pl.kernel(
        _make_sc_body(epw, nch, tail),
        out_type=jax.ShapeDtypeStruct((NC, N, D), jnp.float32),
        mesh=mesh,
        scratch_types=[
            pltpu.VMEM_SHARED((N, D), jnp.float32),
            [pltpu.VMEM((CHUNK,), jnp.int32) for _ in range(2)],
            [pltpu.VMEM((CHUNK,), jnp.int32) for _ in range(4)],
            [pltpu.VMEM((CHUNK, D), jnp.float32) for _ in range(2)],
            [pltpu.VMEM((CHUNK, D), jnp.float32) for _ in range(2)],
            pltpu.VMEM((max(tail, 8),), jnp.int32),
            [pltpu.SemaphoreType.DMA for _ in range(2)],
            [pltpu.SemaphoreType.DMA for _ in range(4)],
            [pltpu.SemaphoreType.DMA for _ in range(2)],
            [pltpu.SemaphoreType.DMA for _ in range(2)],
            [pltpu.SemaphoreType.DMA for _ in range(2)],
        ],
    )
    return kern(node_feats, senders, receivers, coeff)


def _combine_body(p0_ref, p1_ref, out_ref):
    out_ref[...] = (p0_ref[0] + p0_ref[1]) + (p1_ref[0] + p1_ref[1])


def _combine(part0, part1):
    BN = 2000
    return pl.pallas_call(
        _combine_body,
        grid=(N // BN,),
        in_specs=[pl.BlockSpec((NC, BN, D), lambda i: (0, i, 0)),
                  pl.BlockSpec((NC, BN, D), lambda i: (0, i, 0))],
        out_specs=pl.BlockSpec((BN, D), lambda i: (i, 0)),
        out_shape=jax.ShapeDtypeStruct((N, D), jnp.float32),
    )(part0, part1)


def kernel(edge_vectors, node_feats, radial_embeddings, senders, receivers,
           W1, W2, w_sh):
    ev_t = edge_vectors.T
    rad_t = radial_embeddings.T
    wsh = w_sh.reshape(1, 4)
    senders = senders.astype(jnp.int32)
    receivers = receivers.astype(jnp.int32)

    parts = []
    lo = 0
    for ne, be, nch, tail in SLICES:
        hi = lo + ne
        coeff = _coeff(ev_t[:, lo:hi], rad_t[:, lo:hi], W1, W2, wsh, be)
        parts.append(_sc_scatter(node_feats, senders[lo:hi],
                                 receivers[lo:hi], coeff, nch, tail))
        lo = hi
    return _combine(parts[0], parts[1])
